# baseline (speedup 1.0000x reference)
import functools

import jax
import jax.numpy as jnp
from jax import lax
from jax.experimental import pallas as pl
from jax.experimental.pallas import tpu as pltpu

_KS = 512
_DIM = 64


def _pick_w(total):
    for w in (3584, 1792, 1024, 512, 256, 128, 64, 32, 16, 8):
        if total % w == 0:
            return w
    return total


def _vq_body(bs_ref, x_hbm, o_hbm, xbuf, obuf, insem, outsem, *, w, perimg):
    k = pl.program_id(0)
    nb = pl.num_programs(0)
    slot = lax.rem(k, 2)

    def cp_in(blk, sl):
        i = lax.div(blk, perimg)
        j = lax.rem(blk, perimg)
        return pltpu.make_async_copy(
            x_hbm.at[i, :, pl.ds(j * w, w)], xbuf.at[sl], insem.at[sl])

    def cp_out(blk, sl):
        i = lax.div(blk, perimg)
        j = lax.rem(blk, perimg)
        return pltpu.make_async_copy(
            obuf.at[sl], o_hbm.at[i, :, pl.ds(j * w, w)], outsem.at[sl])

    @pl.when(k == 0)
    def _():
        cp_in(k, slot).start()

    @pl.when(k + 1 < nb)
    def _():
        cp_in(k + 1, lax.rem(k + 1, 2)).start()

    cp_in(k, slot).wait()

    b_hi = bs_ref[0:_KS]
    b_lo = bs_ref[_KS:2 * _KS]
    b_lo2 = bs_ref[2 * _KS:3 * _KS]
    xt = xbuf[slot]
    dn = (((0,), (0,)), ((), ()))

    def depth(vt, exact):
        ipT = lax.dot_general(b_hi, vt.astype(jnp.bfloat16),
                              (((1,), (0,)), ((), ())),
                              preferred_element_type=jnp.float32)
        u = jnp.max(ipT, axis=0)
        oh = (ipT == u[None, :]).astype(jnp.bfloat16)
        comp = lax.dot_general(b_hi, oh, dn, preferred_element_type=jnp.float32)
        if exact:
            comp = (comp
                    + lax.dot_general(b_lo, oh, dn, preferred_element_type=jnp.float32)
                    + lax.dot_general(b_lo2, oh, dn, preferred_element_type=jnp.float32))
        return comp * u[None, :]

    comp1 = depth(xt, True)
    res = comp1 + depth(xt - comp1, False)

    @pl.when(k >= 2)
    def _():
        cp_out(k - 2, slot).wait()

    obuf[slot] = res
    cp_out(k, slot).start()

    @pl.when(k == nb - 1)
    def _():
        cp_out(k - 1, lax.rem(k + 1, 2)).wait()
        cp_out(k, slot).wait()


def kernel(x, code_book):
    n, dim, h, wd = x.shape
    total = h * wd
    w = _pick_w(total)
    perimg = total // w
    xr = x.reshape(n, dim, total)
    b_hi = code_book.astype(jnp.bfloat16)
    r1 = code_book - b_hi.astype(jnp.float32)
    b_lo = r1.astype(jnp.bfloat16)
    b_lo2 = (r1 - b_lo.astype(jnp.float32)).astype(jnp.bfloat16)
    bsplit = jnp.concatenate([b_hi, b_lo, b_lo2], axis=0)
    out = pl.pallas_call(
        functools.partial(_vq_body, w=w, perimg=perimg),
        grid=(n * perimg,),
        in_specs=[
            pl.BlockSpec((3 * _KS, _DIM), lambda k: (0, 0)),
            pl.BlockSpec(memory_space=pltpu.MemorySpace.HBM),
        ],
        out_specs=pl.BlockSpec(memory_space=pltpu.MemorySpace.HBM),
        out_shape=jax.ShapeDtypeStruct((n, dim, total), jnp.float32),
        scratch_shapes=[
            pltpu.VMEM((2, dim, w), jnp.float32),
            pltpu.VMEM((2, dim, w), jnp.float32),
            pltpu.SemaphoreType.DMA((2,)),
            pltpu.SemaphoreType.DMA((2,)),
        ],
        compiler_params=pltpu.CompilerParams(
            dimension_semantics=("arbitrary",),
        ),
    )(bsplit, xr)
    return out.reshape(n, dim, h, wd)


# speedup vs baseline: 1.0749x; 1.0749x over previous
import functools

import jax
import jax.numpy as jnp
from jax import lax
from jax.experimental import pallas as pl
from jax.experimental.pallas import tpu as pltpu

_KS = 512
_DIM = 64


def _pick_split(total):
    for halves in (2, 1):
        if total % halves:
            continue
        blk = total // halves
        for w in (1792, 1024, 512, 256, 128, 64, 32, 16, 8):
            if blk % w == 0:
                return blk, w
    return total, total


def _vq_body(bs_ref, x_ref, o_ref, *, w, nsub):
    b_hi = bs_ref[0:_KS]
    b_lo = bs_ref[_KS:2 * _KS]
    b_lo2 = bs_ref[2 * _KS:3 * _KS]
    dn = (((0,), (0,)), ((), ()))

    def depth(vt, exact):
        ipT = lax.dot_general(b_hi, vt.astype(jnp.bfloat16),
                              (((1,), (0,)), ((), ())),
                              preferred_element_type=jnp.float32)
        u = jnp.max(ipT, axis=0)
        oh = (ipT == u[None, :]).astype(jnp.bfloat16)
        comp = lax.dot_general(b_hi, oh, dn, preferred_element_type=jnp.float32)
        if exact:
            comp = (comp
                    + lax.dot_general(b_lo, oh, dn, preferred_element_type=jnp.float32)
                    + lax.dot_general(b_lo2, oh, dn, preferred_element_type=jnp.float32))
        return comp * u[None, :]

    for s in range(nsub):
        xt = x_ref[0, :, s * w:(s + 1) * w]
        comp1 = depth(xt, True)
        o_ref[0, :, s * w:(s + 1) * w] = comp1 + depth(xt - comp1, False)


def kernel(x, code_book):
    n, dim, h, wd = x.shape
    total = h * wd
    blk, w = _pick_split(total)
    nsub = blk // w
    perimg = total // blk
    xr = x.reshape(n, dim, total)
    b_hi = code_book.astype(jnp.bfloat16)
    r1 = code_book - b_hi.astype(jnp.float32)
    b_lo = r1.astype(jnp.bfloat16)
    b_lo2 = (r1 - b_lo.astype(jnp.float32)).astype(jnp.bfloat16)
    bsplit = jnp.concatenate([b_hi, b_lo, b_lo2], axis=0)
    out = pl.pallas_call(
        functools.partial(_vq_body, w=w, nsub=nsub),
        grid=(n, perimg),
        in_specs=[
            pl.BlockSpec((3 * _KS, _DIM), lambda i, j: (0, 0)),
            pl.BlockSpec((1, dim, blk), lambda i, j: (i, 0, j)),
        ],
        out_specs=pl.BlockSpec((1, dim, blk), lambda i, j: (i, 0, j)),
        out_shape=jax.ShapeDtypeStruct((n, dim, total), jnp.float32),
        compiler_params=pltpu.CompilerParams(
            dimension_semantics=("parallel", "parallel"),
        ),
    )(bsplit, xr)
    return out.reshape(n, dim, h, wd)
